# baseline (speedup 1.0000x reference)
import functools

import jax
import jax.numpy as jnp
from jax import lax
from jax.experimental import pallas as pl
from jax.experimental.pallas import tpu as pltpu
from jax.experimental.pallas import tpu_sc as plsc

N = 10000
NP = 10240
E = 320000
G = 64
NC = 2
NS = 16
NW = NC * NS
C = 125
EW = E // (NW * C)
TPR = NP // NS
ZR = 64
WB = 128
BN = 2000
NB = N // BN


def _mesh():
    return plsc.VectorSubcoreMesh(
        core_axis_name="c", subcore_axis_name="s", num_cores=NC)


@functools.partial(
    pl.kernel,
    out_type=jax.ShapeDtypeStruct((NC * NP, 16), jnp.float32),
    mesh=_mesh(),
    scratch_types=[
        pltpu.VMEM_SHARED((NP, 16), jnp.float32),
        pltpu.VMEM((EW, C), jnp.int32),
        pltpu.VMEM((C, 16), jnp.float32),
        pltpu.VMEM((TPR, 16), jnp.float32),
    ],
    compiler_params=pltpu.CompilerParams(use_tc_tiling_on_sc=False),
)
def _sc_deg(ei_hbm, out_hbm, acc_sh, col_v, ones_v, zbuf):
    col_hbm = ei_hbm.at[1]
    cid = lax.axis_index("c")
    sid = lax.axis_index("s")
    wid = sid * NC + cid

    def zrow(i, _):
        zbuf[i] = jnp.zeros((16,), jnp.float32)
        return 0
    lax.fori_loop(0, TPR, zrow, 0)

    def orow(i, _):
        ones_v[i] = jnp.ones((16,), jnp.float32)
        return 0
    lax.fori_loop(0, C, orow, 0)

    r0 = sid * TPR
    pltpu.sync_copy(zbuf, acc_sh.at[pl.ds(r0, TPR)])
    plsc.subcore_barrier()

    pltpu.sync_copy(col_hbm.at[pl.ds(wid * EW, EW)], col_v)

    def body(j, _):
        pltpu.sync_copy(ones_v, acc_sh.at[col_v.at[j]], add=True)
        return 0
    lax.fori_loop(0, EW, body, 0)

    plsc.subcore_barrier()
    pltpu.sync_copy(acc_sh.at[pl.ds(r0, TPR)],
                    out_hbm.at[pl.ds(cid * NP + r0, TPR)])


IB = 16
NBLK = EW // IB


def _make_sc_edge(D, tiled=False):
    @functools.partial(
        pl.kernel,
        out_type=jax.ShapeDtypeStruct((NC * NP, D), jnp.float32),
        mesh=_mesh(),
        scratch_types=[
            pltpu.VMEM_SHARED((NP, D), jnp.float32),
            pltpu.VMEM((2, IB, C), jnp.int32),
            pltpu.VMEM((2, IB, C), jnp.int32),
            pltpu.VMEM((C, D), jnp.float32),
            pltpu.VMEM((C, D), jnp.float32),
            pltpu.VMEM((ZR, D), jnp.float32),
            pltpu.SemaphoreType.DMA,
            pltpu.SemaphoreType.DMA,
            pltpu.SemaphoreType.DMA,
            pltpu.SemaphoreType.DMA,
        ],
        compiler_params=pltpu.CompilerParams(use_tc_tiling_on_sc=tiled),
    )
    def edge_kernel(ei_hbm, y_hbm, out_hbm,
                    s_sh, row_v, col_v, g0, g1, zbuf,
                    sem0, sem1, semi, semz):
        row_hbm = ei_hbm.at[0]
        col_hbm = ei_hbm.at[1]
        cid = lax.axis_index("c")
        sid = lax.axis_index("s")
        wid = sid * NC + cid

        cb = wid * EW
        pltpu.async_copy(row_hbm.at[pl.ds(cb, IB)], row_v.at[0], sem0)
        pltpu.async_copy(col_hbm.at[pl.ds(cb, IB)], col_v.at[0], sem0)
        pltpu.async_copy(row_hbm.at[pl.ds(cb + IB, IB)], row_v.at[1], semi)
        pltpu.async_copy(col_hbm.at[pl.ds(cb + IB, IB)], col_v.at[1], semi)

        def zrow(i, _):
            for k in range(D // 16):
                zbuf[i, pl.ds(k * 16, 16)] = jnp.zeros((16,), jnp.float32)
            return 0
        lax.fori_loop(0, ZR, zrow, 0)

        r0 = sid * TPR

        for q in range(TPR // ZR):
            pltpu.async_copy(zbuf, s_sh.at[pl.ds(r0 + q * ZR, ZR)], semz)
        for q in range(TPR // ZR):
            pltpu.make_async_copy(zbuf, s_sh.at[pl.ds(r0 + q * ZR, ZR)],
                                  semz).wait()
        plsc.subcore_barrier()

        pltpu.make_async_copy(row_hbm.at[pl.ds(cb, IB)], row_v.at[0],
                              sem0).wait()
        pltpu.make_async_copy(col_hbm.at[pl.ds(cb, IB)], col_v.at[0],
                              sem0).wait()
        pltpu.async_copy(y_hbm.at[row_v.at[0, 0]], g0, sem0)

        for b in range(NBLK):
            p, np_ = b % 2, (b + 1) % 2

            def body(jj, _):
                j0 = 2 * jj
                pltpu.async_copy(y_hbm.at[row_v.at[p, j0 + 1]], g1, sem1)
                pltpu.make_async_copy(y_hbm.at[row_v.at[p, j0]],
                                      g0, sem0).wait()
                pltpu.sync_copy(g0, s_sh.at[col_v.at[p, j0]], add=True)

                @pl.when(j0 + 2 < IB)
                def _():
                    pltpu.async_copy(y_hbm.at[row_v.at[p, j0 + 2]], g0, sem0)
                pltpu.make_async_copy(y_hbm.at[row_v.at[p, j0 + 1]],
                                      g1, sem1).wait()
                pltpu.sync_copy(g1, s_sh.at[col_v.at[p, j0 + 1]], add=True)
                return 0
            lax.fori_loop(0, IB // 2, body, 0)

            if b < NBLK - 1:
                pltpu.make_async_copy(row_hbm.at[pl.ds(cb, IB)],
                                      row_v.at[np_], semi).wait()
                pltpu.make_async_copy(col_hbm.at[pl.ds(cb, IB)],
                                      col_v.at[np_], semi).wait()
                pltpu.async_copy(y_hbm.at[row_v.at[np_, 0]], g0, sem0)
                if b < NBLK - 2:
                    nb = cb + (b + 2) * IB
                    pltpu.async_copy(row_hbm.at[pl.ds(nb, IB)],
                                     row_v.at[p], semi)
                    pltpu.async_copy(col_hbm.at[pl.ds(nb, IB)],
                                     col_v.at[p], semi)

        plsc.subcore_barrier()
        base = cid * NP + r0
        for q in range(TPR // WB):
            pltpu.async_copy(s_sh.at[pl.ds(r0 + q * WB, WB)],
                             out_hbm.at[pl.ds(base + q * WB, WB)], semz)
        for q in range(TPR // WB):
            pltpu.make_async_copy(s_sh.at[pl.ds(r0 + q * WB, WB)],
                                  out_hbm.at[pl.ds(base + q * WB, WB)],
                                  semz).wait()
    return edge_kernel


_sc_edge_1 = _make_sc_edge(128, tiled=True)
_sc_edge_2 = _make_sc_edge(64)


def _dis_of(da):
    deg = da[0, :, 0:1] + da[1, :, 0:1] + 1.0
    return lax.rsqrt(deg)


def _tc1a_body(x_ref, w_ref, y_ref):
    y_ref[...] = jnp.dot(x_ref[...], w_ref[...],
                         preferred_element_type=jnp.float32)


def _tc1b_body(xw_ref, da_ref, y_ref):
    y_ref[...] = xw_ref[...] * _dis_of(da_ref[...])


def _tc2_body(s_ref, y1_ref, da_ref, b1_ref, w2_ref, y2_ref):
    dis = _dis_of(da_ref[...])
    s = s_ref[...]
    h = jnp.maximum(dis * (s[0] + s[1] + y1_ref[...]) + b1_ref[...], 0.0)
    y2_ref[...] = jnp.dot(h, w2_ref[...],
                          preferred_element_type=jnp.float32) * dis


def _tc3_body(s_ref, y2_ref, da_ref, b2_ref, batch_ref,
              pooled_ref, counts_ref):
    i = pl.program_id(0)
    dis = _dis_of(da_ref[...])
    s = s_ref[...]
    h = jnp.maximum(dis * (s[0] + s[1] + y2_ref[...]) + b2_ref[...], 0.0)
    bt = batch_ref[0]
    gids = lax.broadcasted_iota(jnp.int32, (G, BN), 0)
    onehot = (gids == bt).astype(jnp.float32)

    @pl.when(i == 0)
    def _():
        pooled_ref[...] = jnp.zeros_like(pooled_ref)
        counts_ref[...] = jnp.zeros_like(counts_ref)

    pooled_ref[...] += jnp.dot(onehot, h, preferred_element_type=jnp.float32)
    counts_ref[...] += jnp.sum(onehot, axis=1, keepdims=True)


_tc1a = pl.pallas_call(
    _tc1a_body,
    grid=(NB,),
    in_specs=[
        pl.BlockSpec((BN, 128), lambda i: (i, 0)),
        pl.BlockSpec((128, 128), lambda i: (0, 0)),
    ],
    out_specs=pl.BlockSpec((BN, 128), lambda i: (i, 0)),
    out_shape=jax.ShapeDtypeStruct((N, 128), jnp.float32),
)

_tc1b = pl.pallas_call(
    _tc1b_body,
    grid=(NB,),
    in_specs=[
        pl.BlockSpec((BN, 128), lambda i: (i, 0)),
        pl.BlockSpec((NC, BN, 16), lambda i: (0, i, 0)),
    ],
    out_specs=pl.BlockSpec((BN, 128), lambda i: (i, 0)),
    out_shape=jax.ShapeDtypeStruct((N, 128), jnp.float32),
)

_tc2 = pl.pallas_call(
    _tc2_body,
    grid=(NB,),
    in_specs=[
        pl.BlockSpec((NC, BN, 128), lambda i: (0, i, 0)),
        pl.BlockSpec((BN, 128), lambda i: (i, 0)),
        pl.BlockSpec((NC, BN, 16), lambda i: (0, i, 0)),
        pl.BlockSpec((1, 128), lambda i: (0, 0)),
        pl.BlockSpec((128, 64), lambda i: (0, 0)),
    ],
    out_specs=pl.BlockSpec((BN, 64), lambda i: (i, 0)),
    out_shape=jax.ShapeDtypeStruct((N, 64), jnp.float32),
)

_tc3 = pl.pallas_call(
    _tc3_body,
    grid=(NB,),
    in_specs=[
        pl.BlockSpec((NC, BN, 64), lambda i: (0, i, 0)),
        pl.BlockSpec((BN, 64), lambda i: (i, 0)),
        pl.BlockSpec((NC, BN, 16), lambda i: (0, i, 0)),
        pl.BlockSpec((1, 64), lambda i: (0, 0)),
        pl.BlockSpec((1, 1, BN), lambda i: (i, 0, 0)),
    ],
    out_specs=[
        pl.BlockSpec((G, 64), lambda i: (0, 0)),
        pl.BlockSpec((G, 1), lambda i: (0, 0)),
    ],
    out_shape=[
        jax.ShapeDtypeStruct((G, 64), jnp.float32),
        jax.ShapeDtypeStruct((G, 1), jnp.float32),
    ],
    compiler_params=pltpu.CompilerParams(
        dimension_semantics=("arbitrary",)),
)


def kernel(x, edge_index, batch, W1, b1, W2, b2):
    ei3 = edge_index.astype(jnp.int32).reshape(2, E // C, C)
    batch3d = batch.astype(jnp.int32).reshape(NB, 1, BN)

    degacc = _sc_deg(ei3).reshape(NC, NP, 16)
    xw1 = _tc1a(x, W1)
    y1 = _tc1b(xw1, degacc)
    s1 = _sc_edge_1(ei3, y1).reshape(NC, NP, 128)
    y2 = _tc2(s1, y1, degacc, b1.reshape(1, 128), W2)
    s2 = _sc_edge_2(ei3, y2).reshape(NC, NP, 64)
    pooled, counts = _tc3(s2, y2, degacc, b2.reshape(1, 64), batch3d)
    return pooled / jnp.maximum(counts, 1.0)


# speedup vs baseline: 1.0044x; 1.0044x over previous
import functools

import jax
import jax.numpy as jnp
from jax import lax
from jax.experimental import pallas as pl
from jax.experimental.pallas import tpu as pltpu
from jax.experimental.pallas import tpu_sc as plsc

N = 10000
NP = 10240
E = 320000
G = 64
NC = 2
NS = 16
NW = NC * NS
C = 125
EW = E // (NW * C)
TPR = NP // NS
ZR = 64
WB = 128
BN = 2000
NB = N // BN


def _mesh():
    return plsc.VectorSubcoreMesh(
        core_axis_name="c", subcore_axis_name="s", num_cores=NC)


@functools.partial(
    pl.kernel,
    out_type=jax.ShapeDtypeStruct((NC * NP, 16), jnp.float32),
    mesh=_mesh(),
    scratch_types=[
        pltpu.VMEM_SHARED((NP, 16), jnp.float32),
        pltpu.VMEM((EW, C), jnp.int32),
        pltpu.VMEM((C, 16), jnp.float32),
        pltpu.VMEM((TPR, 16), jnp.float32),
    ],
    compiler_params=pltpu.CompilerParams(use_tc_tiling_on_sc=False),
)
def _sc_deg(ei_hbm, out_hbm, acc_sh, col_v, ones_v, zbuf):
    col_hbm = ei_hbm.at[1]
    cid = lax.axis_index("c")
    sid = lax.axis_index("s")
    wid = sid * NC + cid

    def zrow(i, _):
        zbuf[i] = jnp.zeros((16,), jnp.float32)
        return 0
    lax.fori_loop(0, TPR, zrow, 0)

    def orow(i, _):
        ones_v[i] = jnp.ones((16,), jnp.float32)
        return 0
    lax.fori_loop(0, C, orow, 0)

    r0 = sid * TPR
    pltpu.sync_copy(zbuf, acc_sh.at[pl.ds(r0, TPR)])
    plsc.subcore_barrier()

    pltpu.sync_copy(col_hbm.at[pl.ds(wid * EW, EW)], col_v)

    def body(j, _):
        pltpu.sync_copy(ones_v, acc_sh.at[col_v.at[j]], add=True)
        return 0
    lax.fori_loop(0, EW, body, 0)

    plsc.subcore_barrier()
    pltpu.sync_copy(acc_sh.at[pl.ds(r0, TPR)],
                    out_hbm.at[pl.ds(cid * NP + r0, TPR)])


IB = 16
NBLK = EW // IB


def _make_sc_edge(D):
    @functools.partial(
        pl.kernel,
        out_type=jax.ShapeDtypeStruct((NC * NP, D), jnp.float32),
        mesh=_mesh(),
        scratch_types=[
            pltpu.VMEM_SHARED((NP, D), jnp.float32),
            pltpu.VMEM((2, IB, C), jnp.int32),
            pltpu.VMEM((2, IB, C), jnp.int32),
            pltpu.VMEM((C, D), jnp.float32),
            pltpu.VMEM((C, D), jnp.float32),
            pltpu.VMEM((ZR, D), jnp.float32),
            pltpu.SemaphoreType.DMA,
            pltpu.SemaphoreType.DMA,
            pltpu.SemaphoreType.DMA,
            pltpu.SemaphoreType.DMA,
        ],
        compiler_params=pltpu.CompilerParams(use_tc_tiling_on_sc=False),
    )
    def edge_kernel(ei_hbm, y_hbm, out_hbm,
                    s_sh, row_v, col_v, g0, g1, zbuf,
                    sem0, sem1, semi, semz):
        row_hbm = ei_hbm.at[0]
        col_hbm = ei_hbm.at[1]
        cid = lax.axis_index("c")
        sid = lax.axis_index("s")
        wid = sid * NC + cid

        cb = wid * EW
        pltpu.async_copy(row_hbm.at[pl.ds(cb, IB)], row_v.at[0], sem0)
        pltpu.async_copy(col_hbm.at[pl.ds(cb, IB)], col_v.at[0], sem0)
        pltpu.async_copy(row_hbm.at[pl.ds(cb + IB, IB)], row_v.at[1], semi)
        pltpu.async_copy(col_hbm.at[pl.ds(cb + IB, IB)], col_v.at[1], semi)

        def zrow(i, _):
            for k in range(D // 16):
                zbuf[i, pl.ds(k * 16, 16)] = jnp.zeros((16,), jnp.float32)
            return 0
        lax.fori_loop(0, ZR, zrow, 0)

        r0 = sid * TPR

        for q in range(TPR // ZR):
            pltpu.async_copy(zbuf, s_sh.at[pl.ds(r0 + q * ZR, ZR)], semz)
        for q in range(TPR // ZR):
            pltpu.make_async_copy(zbuf, s_sh.at[pl.ds(r0 + q * ZR, ZR)],
                                  semz).wait()
        plsc.subcore_barrier()

        pltpu.make_async_copy(row_hbm.at[pl.ds(cb, IB)], row_v.at[0],
                              sem0).wait()
        pltpu.make_async_copy(col_hbm.at[pl.ds(cb, IB)], col_v.at[0],
                              sem0).wait()
        pltpu.async_copy(y_hbm.at[row_v.at[0, 0]], g0, sem0)

        for b in range(NBLK):
            p, np_ = b % 2, (b + 1) % 2

            def body(jj, _):
                j0 = 2 * jj
                pltpu.async_copy(y_hbm.at[row_v.at[p, j0 + 1]], g1, sem1)
                pltpu.make_async_copy(y_hbm.at[row_v.at[p, j0]],
                                      g0, sem0).wait()
                pltpu.sync_copy(g0, s_sh.at[col_v.at[p, j0]], add=True)

                @pl.when(j0 + 2 < IB)
                def _():
                    pltpu.async_copy(y_hbm.at[row_v.at[p, j0 + 2]], g0, sem0)
                pltpu.make_async_copy(y_hbm.at[row_v.at[p, j0 + 1]],
                                      g1, sem1).wait()
                pltpu.sync_copy(g1, s_sh.at[col_v.at[p, j0 + 1]], add=True)
                return 0
            lax.fori_loop(0, IB // 2, body, 0)

            if b < NBLK - 1:
                pltpu.make_async_copy(row_hbm.at[pl.ds(cb, IB)],
                                      row_v.at[np_], semi).wait()
                pltpu.make_async_copy(col_hbm.at[pl.ds(cb, IB)],
                                      col_v.at[np_], semi).wait()
                pltpu.async_copy(y_hbm.at[row_v.at[np_, 0]], g0, sem0)
                if b < NBLK - 2:
                    nb = cb + (b + 2) * IB
                    pltpu.async_copy(row_hbm.at[pl.ds(nb, IB)],
                                     row_v.at[p], semi)
                    pltpu.async_copy(col_hbm.at[pl.ds(nb, IB)],
                                     col_v.at[p], semi)

        plsc.subcore_barrier()
        base = cid * NP + r0
        for q in range(TPR // WB):
            pltpu.async_copy(s_sh.at[pl.ds(r0 + q * WB, WB)],
                             out_hbm.at[pl.ds(base + q * WB, WB)], semz)
        for q in range(TPR // WB):
            pltpu.make_async_copy(s_sh.at[pl.ds(r0 + q * WB, WB)],
                                  out_hbm.at[pl.ds(base + q * WB, WB)],
                                  semz).wait()
    return edge_kernel


_sc_edge_1 = _make_sc_edge(128)
_sc_edge_2 = _make_sc_edge(64)


def _dis_of(da):
    deg = da[0, :, 0:1] + da[1, :, 0:1] + 1.0
    return lax.rsqrt(deg)


def _tc1a_body(x_ref, w_ref, y_ref):
    y_ref[...] = jnp.dot(x_ref[...], w_ref[...],
                         preferred_element_type=jnp.float32)


def _tc1b_body(xw_ref, da_ref, y_ref):
    y_ref[...] = xw_ref[...] * _dis_of(da_ref[...])


def _tc2_body(s_ref, y1_ref, da_ref, b1_ref, w2_ref, y2_ref):
    dis = _dis_of(da_ref[...])
    s = s_ref[...]
    h = jnp.maximum(dis * (s[0] + s[1] + y1_ref[...]) + b1_ref[...], 0.0)
    y2_ref[...] = jnp.dot(h, w2_ref[...],
                          preferred_element_type=jnp.float32) * dis


def _tc3_body(s_ref, y2_ref, da_ref, b2_ref, batch_ref,
              pooled_ref, counts_ref):
    i = pl.program_id(0)
    dis = _dis_of(da_ref[...])
    s = s_ref[...]
    h = jnp.maximum(dis * (s[0] + s[1] + y2_ref[...]) + b2_ref[...], 0.0)
    bt = batch_ref[0]
    gids = lax.broadcasted_iota(jnp.int32, (G, BN), 0)
    onehot = (gids == bt).astype(jnp.float32)

    @pl.when(i == 0)
    def _():
        pooled_ref[...] = jnp.zeros_like(pooled_ref)
        counts_ref[...] = jnp.zeros_like(counts_ref)

    pooled_ref[...] += jnp.dot(onehot, h, preferred_element_type=jnp.float32)
    counts_ref[...] += jnp.sum(onehot, axis=1, keepdims=True)


_tc1a = pl.pallas_call(
    _tc1a_body,
    grid=(NB,),
    in_specs=[
        pl.BlockSpec((BN, 128), lambda i: (i, 0)),
        pl.BlockSpec((128, 128), lambda i: (0, 0)),
    ],
    out_specs=pl.BlockSpec((BN, 128), lambda i: (i, 0)),
    out_shape=jax.ShapeDtypeStruct((N, 128), jnp.float32),
)

_tc1b = pl.pallas_call(
    _tc1b_body,
    grid=(NB,),
    in_specs=[
        pl.BlockSpec((BN, 128), lambda i: (i, 0)),
        pl.BlockSpec((NC, BN, 16), lambda i: (0, i, 0)),
    ],
    out_specs=pl.BlockSpec((BN, 128), lambda i: (i, 0)),
    out_shape=jax.ShapeDtypeStruct((N, 128), jnp.float32),
)

_tc2 = pl.pallas_call(
    _tc2_body,
    grid=(NB,),
    in_specs=[
        pl.BlockSpec((NC, BN, 128), lambda i: (0, i, 0)),
        pl.BlockSpec((BN, 128), lambda i: (i, 0)),
        pl.BlockSpec((NC, BN, 16), lambda i: (0, i, 0)),
        pl.BlockSpec((1, 128), lambda i: (0, 0)),
        pl.BlockSpec((128, 64), lambda i: (0, 0)),
    ],
    out_specs=pl.BlockSpec((BN, 64), lambda i: (i, 0)),
    out_shape=jax.ShapeDtypeStruct((N, 64), jnp.float32),
)

_tc3 = pl.pallas_call(
    _tc3_body,
    grid=(NB,),
    in_specs=[
        pl.BlockSpec((NC, BN, 64), lambda i: (0, i, 0)),
        pl.BlockSpec((BN, 64), lambda i: (i, 0)),
        pl.BlockSpec((NC, BN, 16), lambda i: (0, i, 0)),
        pl.BlockSpec((1, 64), lambda i: (0, 0)),
        pl.BlockSpec((1, 1, BN), lambda i: (i, 0, 0)),
    ],
    out_specs=[
        pl.BlockSpec((G, 64), lambda i: (0, 0)),
        pl.BlockSpec((G, 1), lambda i: (0, 0)),
    ],
    out_shape=[
        jax.ShapeDtypeStruct((G, 64), jnp.float32),
        jax.ShapeDtypeStruct((G, 1), jnp.float32),
    ],
    compiler_params=pltpu.CompilerParams(
        dimension_semantics=("arbitrary",)),
)


def kernel(x, edge_index, batch, W1, b1, W2, b2):
    ei3 = edge_index.astype(jnp.int32).reshape(2, E // C, C)
    batch3d = batch.astype(jnp.int32).reshape(NB, 1, BN)

    degacc = _sc_deg(ei3).reshape(NC, NP, 16)
    xw1 = _tc1a(x, W1)
    y1 = _tc1b(xw1, degacc)
    s1 = _sc_edge_1(ei3, y1).reshape(NC, NP, 128)
    y2 = _tc2(s1, y1, degacc, b1.reshape(1, 128), W2)
    s2 = _sc_edge_2(ei3, y2).reshape(NC, NP, 64)
    pooled, counts = _tc3(s2, y2, degacc, b2.reshape(1, 64), batch3d)
    return pooled / jnp.maximum(counts, 1.0)
